# Initial kernel scaffold; baseline (speedup 1.0000x reference)
import jax
import jax.numpy as jnp
from jax.experimental import pallas as pl


def kernel(x, branch_w_all, branch_b_all, proxy_wnT, proxy_sigma, fc_wnT, fc_sigma):
    raise NotImplementedError("write your pallas kernel here")


# speedup vs baseline: 1.0027x; 1.0027x over previous
import functools

import jax
import jax.numpy as jnp
from jax.experimental import pallas as pl
from jax.experimental.pallas import tpu as pltpu

_OUT_DIM = 128
_PATCH = 16
_CIN = 3
_LANE = 128
_NB_CLASSES = 16
_VMEM_LIMIT_BYTES = 32 * 1024 * 1024


def _fused_kernel(p_ref, w_ref, b_ref, fcw_ref, sig_ref, feat_ref, log_ref,
                  *, np_per_img):
    h = jnp.dot(p_ref[...], w_ref[...], preferred_element_type=jnp.float32)
    h = jax.nn.gelu(h + b_ref[...])

    tm, _ = h.shape
    b_tile = tm // np_per_img
    col = jax.lax.broadcasted_iota(jnp.int32, (b_tile, tm), 1)
    row = jax.lax.broadcasted_iota(jnp.int32, (b_tile, tm), 0)
    pool = jnp.where(col // np_per_img == row,
                     jnp.float32(1.0 / np_per_img), jnp.float32(0.0))
    feats = jnp.dot(pool, h, preferred_element_type=jnp.float32)
    feat_ref[...] = feats

    xn = feats * jax.lax.rsqrt(
        jnp.maximum(jnp.sum(feats * feats, axis=-1, keepdims=True), 1e-24))
    logits = jnp.dot(xn, fcw_ref[...], preferred_element_type=jnp.float32)
    log_ref[...] = sig_ref[0, 0] * logits


def kernel(x, branch_w_all, branch_b_all, proxy_wnT, proxy_sigma, fc_wnT,
           fc_sigma):
    del proxy_wnT, proxy_sigma
    B, C, H, W = x.shape
    gh, gw = H // _PATCH, W // _PATCH
    np_per_img = gh * gw

    p = x.astype(jnp.bfloat16)
    p = p.reshape(B, C, gh, _PATCH, gw, _PATCH)
    p = p.transpose(0, 2, 4, 1, 3, 5)
    patches = p.reshape(B * np_per_img, C * _PATCH * _PATCH)

    K = patches.shape[1]
    N = branch_w_all.shape[1]
    Cp = fc_wnT.shape[1]

    b_tile = 8
    grid = (B // b_tile,)
    tm = b_tile * np_per_img

    body = functools.partial(_fused_kernel, np_per_img=np_per_img)
    feats, logits_pad = pl.pallas_call(
        body,
        out_shape=(jax.ShapeDtypeStruct((B, N), jnp.float32),
                   jax.ShapeDtypeStruct((B, Cp), jnp.float32)),
        grid_spec=pltpu.PrefetchScalarGridSpec(
            num_scalar_prefetch=0,
            grid=grid,
            in_specs=[
                pl.BlockSpec((tm, K), lambda m: (m, 0)),
                pl.BlockSpec((K, N), lambda m: (0, 0)),
                pl.BlockSpec((1, N), lambda m: (0, 0)),
                pl.BlockSpec((N, Cp), lambda m: (0, 0)),
                pl.BlockSpec(memory_space=pltpu.MemorySpace.SMEM,
                             block_shape=(1, 1), index_map=lambda m: (0, 0)),
            ],
            out_specs=(pl.BlockSpec((b_tile, N), lambda m: (m, 0)),
                       pl.BlockSpec((b_tile, Cp), lambda m: (m, 0))),
        ),
        compiler_params=pltpu.CompilerParams(
            dimension_semantics=("parallel",),
            vmem_limit_bytes=_VMEM_LIMIT_BYTES),
    )(patches, branch_w_all, branch_b_all, fc_wnT,
      fc_sigma.reshape(1, 1))

    return {'logits': logits_pad[:, :_NB_CLASSES], 'features': feats}


# speedup vs baseline: 2.8688x; 2.8611x over previous
import functools

import jax
import jax.numpy as jnp
from jax.experimental import pallas as pl
from jax.experimental.pallas import tpu as pltpu

_PATCH = 16
_CIN = 3
_NB_CLASSES = 16
_VMEM_LIMIT_BYTES = 32 * 1024 * 1024


def _extract_kernel(x_ref, o_ref, *, batch, gw):
    cq = _CIN * _PATCH
    a = x_ref[...].reshape(batch, cq, gw * _PATCH)
    a = a.astype(jnp.bfloat16)
    t = jnp.swapaxes(a, 1, 2)
    o_ref[...] = t.reshape(batch, 1, gw, _PATCH, cq)


def _fused_kernel(p_ref, w_ref, b_ref, fcw_ref, sig_ref, feat_ref, log_ref,
                  *, np_per_img):
    h = jnp.dot(p_ref[...], w_ref[...], preferred_element_type=jnp.float32)
    h = jax.nn.gelu(h + b_ref[...])

    tm, _ = h.shape
    b_tile = tm // np_per_img
    col = jax.lax.broadcasted_iota(jnp.int32, (b_tile, tm), 1)
    row = jax.lax.broadcasted_iota(jnp.int32, (b_tile, tm), 0)
    pool = jnp.where(col // np_per_img == row,
                     jnp.float32(1.0 / np_per_img), jnp.float32(0.0))
    feats = jnp.dot(pool, h, preferred_element_type=jnp.float32)
    feat_ref[...] = feats

    xn = feats * jax.lax.rsqrt(
        jnp.maximum(jnp.sum(feats * feats, axis=-1, keepdims=True), 1e-24))
    logits = jnp.dot(xn, fcw_ref[...], preferred_element_type=jnp.float32)
    log_ref[...] = sig_ref[0, 0] * logits


def kernel(x, branch_w_all, branch_b_all, proxy_wnT, proxy_sigma, fc_wnT,
           fc_sigma):
    del proxy_wnT, proxy_sigma
    B, C, H, W = x.shape
    gh, gw = H // _PATCH, W // _PATCH
    np_per_img = gh * gw
    K = C * _PATCH * _PATCH
    N = branch_w_all.shape[1]
    Cp = fc_wnT.shape[1]

    x5 = x.reshape(B, C, gh, _PATCH, W)
    extract = functools.partial(_extract_kernel, batch=B, gw=gw)
    patches5 = pl.pallas_call(
        extract,
        out_shape=jax.ShapeDtypeStruct((B, gh, gw, _PATCH, C * _PATCH),
                                       jnp.bfloat16),
        grid_spec=pltpu.PrefetchScalarGridSpec(
            num_scalar_prefetch=0,
            grid=(gh,),
            in_specs=[pl.BlockSpec((B, C, 1, _PATCH, W),
                                   lambda i: (0, 0, i, 0, 0))],
            out_specs=pl.BlockSpec((B, 1, gw, _PATCH, C * _PATCH),
                                   lambda i: (0, i, 0, 0, 0)),
        ),
        compiler_params=pltpu.CompilerParams(
            dimension_semantics=("parallel",),
            vmem_limit_bytes=_VMEM_LIMIT_BYTES),
    )(x5)
    patches = patches5.reshape(B * np_per_img, K)

    w_perm = (branch_w_all.reshape(C, _PATCH, _PATCH, N)
              .transpose(2, 0, 1, 3).reshape(K, N))

    b_tile = 8
    tm = b_tile * np_per_img
    body = functools.partial(_fused_kernel, np_per_img=np_per_img)
    feats, logits_pad = pl.pallas_call(
        body,
        out_shape=(jax.ShapeDtypeStruct((B, N), jnp.float32),
                   jax.ShapeDtypeStruct((B, Cp), jnp.float32)),
        grid_spec=pltpu.PrefetchScalarGridSpec(
            num_scalar_prefetch=0,
            grid=(B // b_tile,),
            in_specs=[
                pl.BlockSpec((tm, K), lambda m: (m, 0)),
                pl.BlockSpec((K, N), lambda m: (0, 0)),
                pl.BlockSpec((1, N), lambda m: (0, 0)),
                pl.BlockSpec((N, Cp), lambda m: (0, 0)),
                pl.BlockSpec(memory_space=pltpu.MemorySpace.SMEM,
                             block_shape=(1, 1), index_map=lambda m: (0, 0)),
            ],
            out_specs=(pl.BlockSpec((b_tile, N), lambda m: (m, 0)),
                       pl.BlockSpec((b_tile, Cp), lambda m: (m, 0))),
        ),
        compiler_params=pltpu.CompilerParams(
            dimension_semantics=("parallel",),
            vmem_limit_bytes=_VMEM_LIMIT_BYTES),
    )(patches, w_perm, branch_b_all, fc_wnT, fc_sigma.reshape(1, 1))

    return {'logits': logits_pad[:, :_NB_CLASSES], 'features': feats}
